# baseline (speedup 1.0000x reference)
import functools

import numpy as np
import jax
import jax.numpy as jnp
from jax import lax
from jax.experimental import pallas as pl
from jax.experimental.pallas import tpu as pltpu
from jax.experimental.pallas import tpu_sc as plsc

_LMAX = 3
_CMAX = 128
_SRC = (_LMAX + 1) * _CMAX
_DST = (_LMAX + 1) ** 2 * _CMAX
_N = 50000

_NC, _NS = 2, 16
_NW = _NC * _NS
_R = 16
_NBLK = _N // _R
_BLK_PER_W = -(-_NBLK // _NW)
_IN_BLK = _R * _SRC
_OUT_BLK = _R * _DST

_IDX_NP = np.array(
    [l * _CMAX + c
     for l in range(_LMAX + 1)
     for c in range(_CMAX)
     for _ in range(2 * l + 1)],
    dtype=np.int32,
)
_TIDX_NP = (_IDX_NP // 128) * 1024 + _IDX_NP % 128


def _make_expand():
    mesh = plsc.VectorSubcoreMesh(
        core_axis_name="c", subcore_axis_name="s",
        num_cores=_NC, num_subcores=_NS)

    @functools.partial(
        pl.kernel,
        out_type=jax.ShapeDtypeStruct((_NBLK * _OUT_BLK,), jnp.float32),
        mesh=mesh,
        scratch_types=[
            pltpu.VMEM((_DST,), jnp.int32),
            pltpu.VMEM((_IN_BLK,), jnp.float32),
            pltpu.VMEM((_IN_BLK,), jnp.float32),
            pltpu.VMEM((_OUT_BLK,), jnp.float32),
            pltpu.VMEM((_OUT_BLK,), jnp.float32),
            pltpu.SemaphoreType.DMA,
            pltpu.SemaphoreType.DMA,
            pltpu.SemaphoreType.DMA,
            pltpu.SemaphoreType.DMA,
        ],
        compiler_params=pltpu.CompilerParams(
            use_tc_tiling_on_sc=False, needs_layout_passes=False),
    )
    def expand(x_hbm, idx_hbm, out_hbm,
               idx_v, in_v0, in_v1, out_v0, out_v1,
               in_s0, in_s1, out_s0, out_s1):
        wid = lax.axis_index("s") * _NC + lax.axis_index("c")
        in_bufs, out_bufs = (in_v0, in_v1), (out_v0, out_v1)
        in_sems, out_sems = (in_s0, in_s1), (out_s0, out_s1)
        pltpu.sync_copy(idx_hbm, idx_v)
        roffs = [jnp.full((16,), (r // 8) * 4096 + (r % 8) * 128, jnp.int32)
                 for r in range(_R)]
        soffs = [(r // 8) * 16384 + (r % 8) * 128 for r in range(_R)]

        for p in range(2):
            b = wid + p * _NW

            @pl.when(b < _NBLK)
            def _(b=b, p=p):
                pltpu.async_copy(
                    x_hbm.at[pl.ds(b * _IN_BLK, _IN_BLK)],
                    in_bufs[p], in_sems[p])

        def iter_body(i, carry):
            for p in range(2):
                k = i * 2 + p
                b = wid + k * _NW

                @pl.when(b < _NBLK)
                def _(k=k, b=b, p=p):
                    pltpu.make_async_copy(
                        x_hbm.at[pl.ds(b * _IN_BLK, _IN_BLK)], in_bufs[p],
                        in_sems[p]).wait()

                    @pl.when(k >= 2)
                    def _():
                        pltpu.make_async_copy(
                            out_bufs[p],
                            out_hbm.at[pl.ds(b * _OUT_BLK, _OUT_BLK)],
                            out_sems[p]).wait()

                    @plsc.parallel_loop(0, _DST // 16, 1, unroll=4)
                    def g_body(g):
                        base = g * 16
                        soff = base + (g >> 3) * 896
                        tg = idx_v[pl.ds(base, 16)]
                        for r in range(_R):
                            out_bufs[p][pl.ds(soff + soffs[r], 16)] = (
                                plsc.load_gather(in_bufs[p], [tg + roffs[r]]))

                    pltpu.async_copy(
                        out_bufs[p],
                        out_hbm.at[pl.ds(b * _OUT_BLK, _OUT_BLK)],
                        out_sems[p])
                    b2 = wid + (k + 2) * _NW

                    @pl.when(b2 < _NBLK)
                    def _():
                        pltpu.async_copy(
                            x_hbm.at[pl.ds(b2 * _IN_BLK, _IN_BLK)],
                            in_bufs[p], in_sems[p])

            return carry

        lax.fori_loop(0, _BLK_PER_W // 2, iter_body, 0)

        for p in range(2):
            k = _BLK_PER_W - 2 + p
            b = wid + k * _NW

            @pl.when(b < _NBLK)
            def _(b=b, p=p):
                pltpu.make_async_copy(
                    out_bufs[p], out_hbm.at[pl.ds(b * _OUT_BLK, _OUT_BLK)],
                    out_sems[p]).wait()

    return expand


_EXPAND = _make_expand()


def kernel(x):
    xt = x.reshape(_N // 8, 8, _SRC // 128, 128)
    xt = xt.transpose(0, 2, 1, 3).reshape(-1)
    outf = _EXPAND(xt, jnp.asarray(_TIDX_NP))
    out = outf.reshape(_N // 8, _DST // 128, 8, 128)
    return out.transpose(0, 2, 1, 3).reshape(_N, _DST)


# speedup vs baseline: 1.0214x; 1.0214x over previous
import functools

import numpy as np
import jax
import jax.numpy as jnp
from jax import lax
from jax.experimental import pallas as pl
from jax.experimental.pallas import tpu as pltpu
from jax.experimental.pallas import tpu_sc as plsc

_LMAX = 3
_CMAX = 128
_SRC = (_LMAX + 1) * _CMAX
_DST = (_LMAX + 1) ** 2 * _CMAX
_N = 50000

_NC, _NS = 2, 16
_NW = _NC * _NS
_R = 16
_NBLK = _N // _R
_BLK_PER_W = -(-_NBLK // _NW)
_IN_BLK = _R * _SRC
_OUT_BLK = _R * _DST

_IDX_NP = np.array(
    [l * _CMAX + c
     for l in range(_LMAX + 1)
     for c in range(_CMAX)
     for _ in range(2 * l + 1)],
    dtype=np.int32,
)
_TIDX_NP = (_IDX_NP // 128) * 1024 + _IDX_NP % 128


def _make_expand():
    mesh = plsc.VectorSubcoreMesh(
        core_axis_name="c", subcore_axis_name="s",
        num_cores=_NC, num_subcores=_NS)

    @functools.partial(
        pl.kernel,
        out_type=jax.ShapeDtypeStruct((_NBLK * _OUT_BLK,), jnp.float32),
        mesh=mesh,
        scratch_types=[
            pltpu.VMEM((_DST,), jnp.int32),
            pltpu.VMEM((_IN_BLK,), jnp.float32),
            pltpu.VMEM((_IN_BLK,), jnp.float32),
            pltpu.VMEM((_IN_BLK,), jnp.float32),
            pltpu.VMEM((_OUT_BLK,), jnp.float32),
            pltpu.VMEM((_OUT_BLK,), jnp.float32),
            pltpu.VMEM((_OUT_BLK,), jnp.float32),
            pltpu.SemaphoreType.DMA,
            pltpu.SemaphoreType.DMA,
            pltpu.SemaphoreType.DMA,
            pltpu.SemaphoreType.DMA,
            pltpu.SemaphoreType.DMA,
            pltpu.SemaphoreType.DMA,
        ],
        compiler_params=pltpu.CompilerParams(
            use_tc_tiling_on_sc=False, needs_layout_passes=False),
    )
    def expand(x_hbm, idx_hbm, out_hbm,
               idx_v, in_v0, in_v1, in_v2, out_v0, out_v1, out_v2,
               in_s0, in_s1, in_s2, out_s0, out_s1, out_s2):
        wid = lax.axis_index("s") * _NC + lax.axis_index("c")
        in_bufs, out_bufs = (in_v0, in_v1, in_v2), (out_v0, out_v1, out_v2)
        in_sems, out_sems = (in_s0, in_s1, in_s2), (out_s0, out_s1, out_s2)
        pltpu.sync_copy(idx_hbm, idx_v)
        roffs = [jnp.full((16,), (r // 8) * 4096 + (r % 8) * 128, jnp.int32)
                 for r in range(_R)]
        soffs = [(r // 8) * 16384 + (r % 8) * 128 for r in range(_R)]

        for p in range(2):
            b = wid + p * _NW

            @pl.when(b < _NBLK)
            def _(b=b, p=p):
                pltpu.async_copy(
                    x_hbm.at[pl.ds(b * _IN_BLK, _IN_BLK)],
                    in_bufs[p], in_sems[p])

        def iter_body(i, carry):
            for q in range(3):
                k = i * 3 + q
                p = q
                b = wid + k * _NW

                @pl.when(b < _NBLK)
                def _(k=k, b=b, p=p):
                    pltpu.make_async_copy(
                        x_hbm.at[pl.ds(b * _IN_BLK, _IN_BLK)], in_bufs[p],
                        in_sems[p]).wait()

                    @pl.when(k >= 3)
                    def _():
                        pltpu.make_async_copy(
                            out_bufs[p],
                            out_hbm.at[pl.ds(b * _OUT_BLK, _OUT_BLK)],
                            out_sems[p]).wait()

                    b2 = wid + (k + 2) * _NW
                    p2 = (q + 2) % 3

                    @pl.when(b2 < _NBLK)
                    def _():
                        pltpu.async_copy(
                            x_hbm.at[pl.ds(b2 * _IN_BLK, _IN_BLK)],
                            in_bufs[p2], in_sems[p2])

                    @plsc.parallel_loop(0, _DST // 16, 1, unroll=2)
                    def g_body(g):
                        base = g * 16
                        soff = base + (g >> 3) * 896
                        tg = idx_v[pl.ds(base, 16)]
                        for r in range(_R):
                            out_bufs[p][pl.ds(soff + soffs[r], 16)] = (
                                plsc.load_gather(in_bufs[p], [tg + roffs[r]]))

                    pltpu.async_copy(
                        out_bufs[p],
                        out_hbm.at[pl.ds(b * _OUT_BLK, _OUT_BLK)],
                        out_sems[p])

            return carry

        lax.fori_loop(0, -(-_BLK_PER_W // 3), iter_body, 0)

        for q in range(3):
            k = _BLK_PER_W - 3 + q
            p = k % 3
            b = wid + k * _NW

            @pl.when(b < _NBLK)
            def _(b=b, p=p):
                pltpu.make_async_copy(
                    out_bufs[p], out_hbm.at[pl.ds(b * _OUT_BLK, _OUT_BLK)],
                    out_sems[p]).wait()

    return expand


_EXPAND = _make_expand()


def kernel(x):
    xt = x.reshape(_N // 8, 8, _SRC // 128, 128)
    xt = xt.transpose(0, 2, 1, 3).reshape(-1)
    outf = _EXPAND(xt, jnp.asarray(_TIDX_NP))
    out = outf.reshape(_N // 8, _DST // 128, 8, 128)
    return out.transpose(0, 2, 1, 3).reshape(_N, _DST)
